# baseline (speedup 1.0000x reference)
import functools

import jax
import jax.numpy as jnp
from jax import lax
from jax.experimental import pallas as pl
from jax.experimental.pallas import tpu as pltpu
from jax.experimental.pallas import tpu_sc as plsc

B, F, V, D = 16384, 26, 100000, 16
T = F * D
BF = B * F
NW = 32
PER_W = BF // NW
CH = 1024
NCH = PER_W // CH

BT = 1024


def _sc_gather_body(table, idxs, out, idx0, idx1, rows0, rows1,
                    gsem0, gsem1, osem0, osem1):
    wid = lax.axis_index("s") * 2 + lax.axis_index("c")
    base = wid * PER_W
    idxv = (idx0, idx1)
    rowsv = (rows0, rows1)
    gsem = (gsem0, gsem1)
    osem = (osem0, osem1)

    copies_g = [None] * NCH
    copies_o = [None] * NCH
    pltpu.sync_copy(idxs.at[pl.ds(base, CH)], idx0)
    copies_g[0] = pltpu.async_copy(table.at[idx0], rows0, gsem0)
    for j in range(NCH):
        b = j & 1
        if j + 1 < NCH:
            nb = b ^ 1
            if j >= 1:
                copies_o[j - 1].wait()
            pltpu.sync_copy(idxs.at[pl.ds(base + (j + 1) * CH, CH)], idxv[nb])
            copies_g[j + 1] = pltpu.async_copy(
                table.at[idxv[nb]], rowsv[nb], gsem[nb])
        copies_g[j].wait()
        copies_o[j] = pltpu.async_copy(
            rowsv[b], out.at[pl.ds(base + j * CH, CH)], osem[b])
    copies_o[NCH - 2].wait()
    copies_o[NCH - 1].wait()


@functools.partial(jax.jit, static_argnums=())
def _sc_gather(table_flat, idx_flat):
    mesh = plsc.VectorSubcoreMesh(core_axis_name="c", subcore_axis_name="s")
    k = functools.partial(
        pl.kernel,
        mesh=mesh,
        out_type=jax.ShapeDtypeStruct((BF, D), jnp.float32),
        scratch_types=[
            pltpu.VMEM((CH,), jnp.int32),
            pltpu.VMEM((CH,), jnp.int32),
            pltpu.VMEM((CH, D), jnp.float32),
            pltpu.VMEM((CH, D), jnp.float32),
            pltpu.SemaphoreType.DMA,
            pltpu.SemaphoreType.DMA,
            pltpu.SemaphoreType.DMA,
            pltpu.SemaphoreType.DMA,
        ],
        compiler_params=pltpu.CompilerParams(use_tc_tiling_on_sc=False),
    )(_sc_gather_body)
    return k(table_flat, idx_flat)


def _tc_body(h_ref, W1_ref, b1_ref, W2_ref, b2_ref, Wfm_ref, Wsm_ref,
             consts_ref, out_ref):
    h = h_ref[...]
    dn = (((1,), (0,)), ((), ()))
    m1 = jnp.maximum(
        lax.dot_general(h, W1_ref[...], dn,
                        preferred_element_type=jnp.float32) + b1_ref[...], 0.0)
    ss = lax.dot_general(h, Wsm_ref[...], dn,
                         preferred_element_type=jnp.float32)
    cv = consts_ref[...]
    probs = []
    for t in range(3):
        m1t = m1[:, t * 256:(t + 1) * 256]
        m2 = jnp.maximum(
            lax.dot_general(m1t, W2_ref[t], dn,
                            preferred_element_type=jnp.float32)
            + b2_ref[t][None, :], 0.0)
        s = jnp.sum(m2 * Wfm_ref[t][None, :], axis=1, keepdims=True)
        a = ss[:, 2 * t:2 * t + 1]
        c = ss[:, 2 * t + 1:2 * t + 2]
        logit = a * c + c + s + cv[0, t]
        probs.append(jax.nn.sigmoid(logit))
    cvr, cf, ctr = probs
    ctcvr = cvr * ctr
    res = jnp.concatenate([cvr, cf, ctr, ctcvr], axis=1)
    out_ref[...] = jnp.clip(res, 1e-15, 1.0 - 1e-15)


@jax.jit
def _tc_dense(h, W1all, b1all, W2all, b2all, Wfm, Wsm, consts):
    full = lambda shape: pl.BlockSpec(shape, lambda i: (0,) * len(shape))
    return pl.pallas_call(
        _tc_body,
        grid=(B // BT,),
        in_specs=[
            pl.BlockSpec((BT, T), lambda i: (i, 0)),
            full((T, 768)),
            full((1, 768)),
            full((3, 256, 128)),
            full((3, 128)),
            full((3, 128)),
            full((T, 8)),
            full((1, 8)),
        ],
        out_specs=pl.BlockSpec((BT, 4), lambda i: (i, 0)),
        out_shape=jax.ShapeDtypeStruct((B, 4), jnp.float32),
        compiler_params=pltpu.CompilerParams(
            dimension_semantics=("parallel",)),
    )(h, W1all, b1all, W2all, b2all, Wfm, Wsm, consts)


def kernel(x, emb_tables,
           cvr_wc, cvr_bc, cvr_W1, cvr_b1, cvr_W2, cvr_b2, cvr_Wf, cvr_bf,
           cf_wc, cf_bc, cf_W1, cf_b1, cf_W2, cf_b2, cf_Wf, cf_bf,
           ctr_wc, ctr_bc, ctr_W1, ctr_b1, ctr_W2, ctr_b2, ctr_Wf, ctr_bf):
    idx_flat = (x.astype(jnp.int32)
                + (jnp.arange(F, dtype=jnp.int32) * V)[None, :]).reshape(-1)
    table_flat = emb_tables.reshape(F * V, D)

    W1all = jnp.concatenate([cvr_W1, cf_W1, ctr_W1], axis=1)
    b1all = jnp.concatenate([cvr_b1, cf_b1, ctr_b1]).reshape(1, 768)
    W2all = jnp.stack([cvr_W2, cf_W2, ctr_W2])
    b2all = jnp.stack([cvr_b2, cf_b2, ctr_b2])
    Wfm = jnp.stack([cvr_Wf[T:, 0], cf_Wf[T:, 0], ctr_Wf[T:, 0]])
    zcol = jnp.zeros((T, 1), jnp.float32)
    Wsm = jnp.concatenate(
        [cvr_wc, cvr_Wf[:T], cf_wc, cf_Wf[:T], ctr_wc, ctr_Wf[:T],
         zcol, zcol], axis=1)
    consts = jnp.stack(
        [jnp.dot(cvr_bc, cvr_Wf[:T, 0]) + cvr_bf[0],
         jnp.dot(cf_bc, cf_Wf[:T, 0]) + cf_bf[0],
         jnp.dot(ctr_bc, ctr_Wf[:T, 0]) + ctr_bf[0],
         jnp.float32(0), jnp.float32(0), jnp.float32(0),
         jnp.float32(0), jnp.float32(0)]).reshape(1, 8)

    h = _sc_gather(table_flat, idx_flat).reshape(B, T)

    return _tc_dense(h, W1all, b1all, W2all, b2all, Wfm, Wsm, consts)
